# baseline (speedup 1.0000x reference)
import functools

import jax
import jax.numpy as jnp
from jax import lax
from jax.experimental import pallas as pl
from jax.experimental.pallas import tpu as pltpu
from jax.experimental.pallas import tpu_sc as plsc

L = 16


def _make_sc_embed(N, D, n_workers, chunk):
    assert N % (n_workers * chunk) == 0
    per_w = N // n_workers
    n_chunks = per_w // chunk
    assert n_chunks % 2 == 0
    mesh = plsc.VectorSubcoreMesh(core_axis_name="c", subcore_axis_name="s")

    @functools.partial(
        pl.kernel,
        mesh=mesh,
        compiler_params=pltpu.CompilerParams(use_tc_tiling_on_sc=False),
        out_type=[
            jax.ShapeDtypeStruct((N, D), jnp.float32),
            jax.ShapeDtypeStruct((N,), jnp.float32),
        ],
        scratch_types=[
            pltpu.VMEM((chunk,), jnp.int32),
            pltpu.VMEM((chunk,), jnp.int32),
            pltpu.VMEM((chunk, D), jnp.float32),
            pltpu.VMEM((chunk, D), jnp.float32),
            pltpu.VMEM((chunk,), jnp.float32),
            pltpu.VMEM((chunk,), jnp.float32),
            pltpu.SemaphoreType.DMA,
            pltpu.SemaphoreType.DMA,
            pltpu.SemaphoreType.DMA,
            pltpu.SemaphoreType.DMA,
        ],
    )
    def sc_embed(
        idx_hbm, table_hbm, out_hbm, mask_hbm,
        idx0, idx1, rows0, rows1, msk0, msk1,
        sg0, sg1, so0, so1,
    ):
        idx_v = [idx0, idx1]
        rows_v = [rows0, rows1]
        mask_v = [msk0, msk1]
        sem_g = [sg0, sg1]
        sem_o = [so0, so1]
        wid = lax.axis_index("s") * 2 + lax.axis_index("c")
        base = wid * per_w

        def wait_out(s):
            pltpu.make_async_copy(out_hbm.at[pl.ds(0, chunk)], rows_v[s], sem_o[s]).wait()
            pltpu.make_async_copy(mask_hbm.at[pl.ds(0, chunk)], mask_v[s], sem_o[s]).wait()

        pltpu.sync_copy(idx_hbm.at[pl.ds(base, chunk)], idx0)
        pltpu.async_copy(table_hbm.at[idx0], rows0, sg0)

        def pair_body(p, carry):
            for s in (0, 1):
                g = p * 2 + s
                o = 1 - s

                @pl.when(g + 1 < n_chunks)
                def _prefetch():
                    @pl.when(g >= 1)
                    def _drain():
                        wait_out(o)

                    off_n = base + (g + 1) * chunk
                    pltpu.sync_copy(idx_hbm.at[pl.ds(off_n, chunk)], idx_v[o])
                    pltpu.async_copy(table_hbm.at[idx_v[o]], rows_v[o], sem_g[o])

                def mask_body(i, acc):
                    iv = idx_v[s][pl.ds(i * L, L)]
                    zm = iv == 0
                    mask_v[s][pl.ds(i * L, L)] = jnp.where(zm, 0.0, 1.0)
                    return acc | jnp.where(zm, 1, 0)

                acc = lax.fori_loop(
                    0, chunk // L, mask_body, jnp.zeros((L,), jnp.int32)
                )
                anyz = acc[0]
                for j in range(1, L):
                    anyz = anyz | acc[j]

                pltpu.make_async_copy(
                    table_hbm.at[pl.ds(0, chunk)], rows_v[s], sem_g[s]
                ).wait()

                @pl.when(anyz != 0)
                def _fix():
                    def fix_body(i, c):
                        mv = mask_v[s][pl.ds(i * L, L)]
                        gz = mv[0]
                        for j in range(1, L):
                            gz = jnp.minimum(gz, mv[j])

                        @pl.when(gz == 0.0)
                        def _():
                            for j in range(L):
                                r = i * L + j
                                rows_v[s][r] = rows_v[s][r] * jnp.full((L,), mv[j])

                        return c

                    lax.fori_loop(0, chunk // L, fix_body, 0)

                off = base + g * chunk
                pltpu.async_copy(rows_v[s], out_hbm.at[pl.ds(off, chunk)], sem_o[s])
                pltpu.async_copy(mask_v[s], mask_hbm.at[pl.ds(off, chunk)], sem_o[s])
            return carry

        lax.fori_loop(0, n_chunks // 2, pair_body, 0)
        wait_out(0)
        wait_out(1)

    return sc_embed


def kernel(x, table):
    b, t, time = x.shape
    v, d = table.shape
    n = b * t * time
    flat = x.reshape(n).astype(jnp.int32)
    emb_flat, mask_flat = _make_sc_embed(n, d, n_workers=32, chunk=1600)(flat, table)
    return emb_flat.reshape(b, t, time, d), mask_flat.reshape(b, t, time)


# speedup vs baseline: 4.5779x; 4.5779x over previous
import functools

import jax
import jax.numpy as jnp
from jax import lax
from jax.experimental import pallas as pl
from jax.experimental.pallas import tpu as pltpu
from jax.experimental.pallas import tpu_sc as plsc

L = 16
NW = 32


def _make_sc_embed(TU, B, D):
    n_iter = (TU + NW - 1) // NW
    eg_n, bg_n = D // 8, B // 128
    mesh = plsc.VectorSubcoreMesh(core_axis_name="c", subcore_axis_name="s")

    @functools.partial(
        pl.kernel,
        mesh=mesh,
        compiler_params=pltpu.CompilerParams(use_tc_tiling_on_sc=False),
        out_type=[
            jax.ShapeDtypeStruct((TU, eg_n, bg_n, 8, 128), jnp.float32),
            jax.ShapeDtypeStruct((TU, B), jnp.float32),
        ],
        scratch_types=[
            pltpu.VMEM((B,), jnp.int32),
            pltpu.VMEM((B,), jnp.int32),
            pltpu.VMEM((B, D), jnp.float32),
            pltpu.VMEM((B, D), jnp.float32),
            pltpu.VMEM((eg_n, bg_n, 8, 128), jnp.float32),
            pltpu.VMEM((eg_n, bg_n, 8, 128), jnp.float32),
            pltpu.VMEM((B,), jnp.float32),
            pltpu.VMEM((B,), jnp.float32),
            pltpu.SemaphoreType.DMA,
            pltpu.SemaphoreType.DMA,
            pltpu.SemaphoreType.DMA,
            pltpu.SemaphoreType.DMA,
        ],
    )
    def sc_embed(
        idx_hbm, table_hbm, out_hbm, mask_hbm,
        idx0, idx1, rows0, rows1, pb0, pb1, msk0, msk1,
        sg0, sg1, so0, so1,
    ):
        idx_v = [idx0, idx1]
        rows_v = [rows0, rows1]
        pb_v = [pb0, pb1]
        mask_v = [msk0, msk1]
        sem_g = [sg0, sg1]
        sem_o = [so0, so1]
        wid = lax.axis_index("s") * 2 + lax.axis_index("c")
        n_mine = (TU - wid + NW - 1) // NW

        lane = lax.iota(jnp.int32, L)
        kmask = {k: (lane & k) == 0 for k in (1, 2, 4, 8)}
        rot_r_idx = {k: (lane - k) & (L - 1) for k in (1, 2, 4, 8)}
        rot_l_idx = {k: (lane + k) & (L - 1) for k in (1, 2, 4, 8)}

        def _perm(v, idx):
            dnums = lax.GatherDimensionNumbers(
                offset_dims=(), collapsed_slice_dims=(0,), start_index_map=(0,)
            )
            return lax.gather(
                v, idx[:, None], dnums, (1,),
                mode=lax.GatherScatterMode.PROMISE_IN_BOUNDS,
            )

        def start_unit(j, s):
            tu = wid + j * NW

            @pl.when(tu < TU)
            def _():
                pltpu.sync_copy(idx_hbm.at[tu], idx_v[s])
                pltpu.async_copy(table_hbm.at[idx_v[s]], rows_v[s], sem_g[s])

        def wait_out(s):
            pltpu.make_async_copy(out_hbm.at[0], pb_v[s], sem_o[s]).wait()
            pltpu.make_async_copy(mask_hbm.at[0], mask_v[s], sem_o[s]).wait()

        start_unit(0, 0)

        def unit_body(j, carry):
            for s in (0, 1):
                jj = j * 2 + s
                tu = wid + jj * NW

                @pl.when(jj + 1 < n_mine)
                def _pf():
                    @pl.when(jj >= 1)
                    def _():
                        wait_out(1 - s)

                    start_unit(jj + 1, 1 - s)

                @pl.when(tu < TU)
                def _work():
                    pltpu.make_async_copy(
                        table_hbm.at[pl.ds(0, B)], rows_v[s], sem_g[s]
                    ).wait()

                    def blk(i, c):
                        b0 = i * L
                        iv = idx_v[s][pl.ds(b0, L)]
                        m = jnp.where(iv == 0, 0.0, 1.0)
                        mask_v[s][pl.ds(b0, L)] = m

                        vs = [rows_v[s][b0 + r] for r in range(L)]
                        for k in (1, 2, 4, 8):
                            nvs = list(vs)
                            km = kmask[k]
                            for a0 in range(L):
                                if a0 & k:
                                    continue
                                p = a0 | k
                                a, b = vs[a0], vs[p]
                                nvs[a0] = jnp.where(km, a, _perm(b, rot_r_idx[k]))
                                nvs[p] = jnp.where(km, _perm(a, rot_l_idx[k]), b)
                            vs = nvs

                        bg = i >> 3
                        boff = (i & 7) * L
                        for e in range(D):
                            pb_v[s][e // 8, bg, e % 8, pl.ds(boff, L)] = vs[e] * m
                        return c

                    lax.fori_loop(0, B // L, blk, 0)

                    pltpu.async_copy(pb_v[s], out_hbm.at[tu], sem_o[s])
                    pltpu.async_copy(mask_v[s], mask_hbm.at[tu], sem_o[s])
            return carry

        lax.fori_loop(0, (n_iter + 1) // 2, unit_body, 0)

        @pl.when(n_mine >= 2)
        def _():
            wait_out(1)

        wait_out(0)

    return sc_embed


def kernel(x, table):
    b, t, time = x.shape
    v, d = table.shape
    tu = t * time
    xk = jnp.transpose(x, (1, 2, 0)).reshape(tu, b).astype(jnp.int32)
    A, maskA = _make_sc_embed(tu, b, d)(xk, table)
    emb = (
        A.reshape(t, time, d // 8, b // 128, 8, 128)
        .transpose(3, 5, 0, 1, 2, 4)
        .reshape(b, t, time, d)
    )
    mask = maskA.reshape(t, time, b).transpose(2, 0, 1)
    return emb, mask


# speedup vs baseline: 4.5912x; 1.0029x over previous
import functools

import jax
import jax.numpy as jnp
from jax import lax
from jax.experimental import pallas as pl
from jax.experimental.pallas import tpu as pltpu
from jax.experimental.pallas import tpu_sc as plsc

L = 16
NW = 32


def _make_sc_embed(TU, B, D, T, TIME):
    n_iter = (TU + NW - 1) // NW
    eg_n, bg_n = D // 8, B // 128
    ug_n = (TIME + 7) // 8
    mesh = plsc.VectorSubcoreMesh(core_axis_name="c", subcore_axis_name="s")

    @functools.partial(
        pl.kernel,
        mesh=mesh,
        compiler_params=pltpu.CompilerParams(use_tc_tiling_on_sc=False),
        out_type=[
            jax.ShapeDtypeStruct((TU, eg_n, bg_n, 8, 128), jnp.float32),
            jax.ShapeDtypeStruct((T, ug_n, bg_n, 8, 128), jnp.float32),
        ],
        scratch_types=[
            pltpu.VMEM((bg_n, 128), jnp.int32),
            pltpu.VMEM((bg_n, 128), jnp.int32),
            pltpu.VMEM((B, D), jnp.float32),
            pltpu.VMEM((B, D), jnp.float32),
            pltpu.VMEM((eg_n, bg_n, 8, 128), jnp.float32),
            pltpu.VMEM((eg_n, bg_n, 8, 128), jnp.float32),
            pltpu.VMEM((bg_n, 128), jnp.float32),
            pltpu.VMEM((bg_n, 128), jnp.float32),
            pltpu.SemaphoreType.DMA,
            pltpu.SemaphoreType.DMA,
            pltpu.SemaphoreType.DMA,
            pltpu.SemaphoreType.DMA,
        ],
    )
    def sc_embed(
        idx_hbm, table_hbm, out_hbm, mask_hbm,
        idx0, idx1, rows0, rows1, pb0, pb1, msk0, msk1,
        sg0, sg1, so0, so1,
    ):
        idx_v = [idx0, idx1]
        rows_v = [rows0, rows1]
        pb_v = [pb0, pb1]
        mask_v = [msk0, msk1]
        sem_g = [sg0, sg1]
        sem_o = [so0, so1]
        wid = lax.axis_index("s") * 2 + lax.axis_index("c")
        n_mine = (TU - wid + NW - 1) // NW

        lane = lax.iota(jnp.int32, L)
        kmask = {k: (lane & k) == 0 for k in (1, 2, 4, 8)}
        rot_r_idx = {k: (lane - k) & (L - 1) for k in (1, 2, 4, 8)}
        rot_l_idx = {k: (lane + k) & (L - 1) for k in (1, 2, 4, 8)}

        def _perm(v, idx):
            dnums = lax.GatherDimensionNumbers(
                offset_dims=(), collapsed_slice_dims=(0,), start_index_map=(0,)
            )
            return lax.gather(
                v, idx[:, None], dnums, (1,),
                mode=lax.GatherScatterMode.PROMISE_IN_BOUNDS,
            )

        def start_unit(j, s):
            tu = wid + j * NW

            @pl.when(tu < TU)
            def _():
                tt = tu // TIME
                u = tu % TIME
                pltpu.sync_copy(idx_hbm.at[tt, u // 8, :, u % 8, :], idx_v[s])
                for bg in range(bg_n):
                    pltpu.async_copy(
                        table_hbm.at[idx_v[s].at[bg]],
                        rows_v[s].at[pl.ds(bg * 128, 128)],
                        sem_g[s],
                    )

        def wait_out(s):
            pltpu.make_async_copy(out_hbm.at[0], pb_v[s], sem_o[s]).wait()
            pltpu.make_async_copy(mask_hbm.at[0, 0, :, 0, :], mask_v[s], sem_o[s]).wait()

        start_unit(0, 0)

        def unit_body(j, carry):
            for s in (0, 1):
                jj = j * 2 + s
                tu = wid + jj * NW

                @pl.when(jj + 1 < n_mine)
                def _pf():
                    @pl.when(jj >= 1)
                    def _():
                        wait_out(1 - s)

                    start_unit(jj + 1, 1 - s)

                @pl.when(tu < TU)
                def _work():
                    pltpu.make_async_copy(
                        table_hbm.at[pl.ds(0, B)], rows_v[s], sem_g[s]
                    ).wait()

                    def blk(i, c):
                        b0 = i * L
                        bgi = i >> 3
                        off = (i & 7) * L
                        iv = idx_v[s][bgi, pl.ds(off, L)]
                        m = jnp.where(iv == 0, 0.0, 1.0)
                        mask_v[s][bgi, pl.ds(off, L)] = m

                        vs = [rows_v[s][b0 + r] for r in range(L)]
                        for k in (1, 2, 4, 8):
                            nvs = list(vs)
                            km = kmask[k]
                            for a0 in range(L):
                                if a0 & k:
                                    continue
                                p = a0 | k
                                a, b = vs[a0], vs[p]
                                nvs[a0] = jnp.where(km, a, _perm(b, rot_r_idx[k]))
                                nvs[p] = jnp.where(km, _perm(a, rot_l_idx[k]), b)
                            vs = nvs

                        for e in range(D):
                            pb_v[s][e // 8, bgi, e % 8, pl.ds(off, L)] = vs[e] * m
                        return c

                    lax.fori_loop(0, B // L, blk, 0)

                    tt = tu // TIME
                    u = tu % TIME
                    pltpu.async_copy(pb_v[s], out_hbm.at[tu], sem_o[s])
                    pltpu.async_copy(
                        mask_v[s], mask_hbm.at[tt, u // 8, :, u % 8, :], sem_o[s]
                    )
            return carry

        lax.fori_loop(0, (n_iter + 1) // 2, unit_body, 0)

        @pl.when(n_mine >= 2)
        def _():
            wait_out(1)

        wait_out(0)

    return sc_embed


def kernel(x, table):
    b, t, time = x.shape
    v, d = table.shape
    tu = t * time
    up = (-time) % 8
    tp = time + up
    xq = jnp.pad(x.astype(jnp.int32), ((0, 0), (0, 0), (0, up)))
    xk = (
        jnp.transpose(xq, (1, 2, 0))
        .reshape(t, tp // 8, 8, b // 128, 128)
        .transpose(0, 1, 3, 2, 4)
    )
    A, maskM = _make_sc_embed(tu, b, d, t, time)(xk, table)
    emb = (
        A.reshape(t, time, d // 8, b // 128, 8, 128)
        .transpose(3, 5, 0, 1, 2, 4)
        .reshape(b, t, time, d)
    )
    mask = (
        maskM.transpose(0, 1, 3, 2, 4)
        .reshape(t, tp, b)
        .transpose(2, 0, 1)[:, :, :time]
    )
    return emb, mask


# speedup vs baseline: 7.6610x; 1.6686x over previous
import functools

import jax
import jax.numpy as jnp
from jax import lax
from jax.experimental import pallas as pl
from jax.experimental.pallas import tpu as pltpu
from jax.experimental.pallas import tpu_sc as plsc

L = 16
NW = 32


def _make_table_linearize(V4, D):
    RG = V4 // 128
    eg_n = D // 8
    mesh = plsc.VectorSubcoreMesh(core_axis_name="c", subcore_axis_name="s")

    @functools.partial(
        pl.kernel,
        mesh=mesh,
        compiler_params=pltpu.CompilerParams(use_tc_tiling_on_sc=False),
        out_type=[jax.ShapeDtypeStruct((V4, D), jnp.float32)],
        scratch_types=[
            pltpu.VMEM((D, 128), jnp.float32),
            pltpu.VMEM((D, 128), jnp.float32),
            pltpu.VMEM((128, D), jnp.float32),
            pltpu.VMEM((128, D), jnp.float32),
            pltpu.SemaphoreType.DMA,
            pltpu.SemaphoreType.DMA,
            pltpu.SemaphoreType.DMA,
            pltpu.SemaphoreType.DMA,
        ],
    )
    def lin(tab4_hbm, out_hbm, t0, t1, o0, o1, si0, si1, so0, so1):
        tile_v = [t0, t1]
        lin_v = [o0, o1]
        sem_i = [si0, si1]
        sem_o = [so0, so1]
        wid = lax.axis_index("s") * 2 + lax.axis_index("c")
        n_mine = (RG - wid + NW - 1) // NW

        lane = lax.iota(jnp.int32, L)
        kmask = {k: (lane & k) == 0 for k in (1, 2, 4, 8)}
        rot_r_idx = {k: (lane - k) & (L - 1) for k in (1, 2, 4, 8)}
        rot_l_idx = {k: (lane + k) & (L - 1) for k in (1, 2, 4, 8)}

        dnums = lax.GatherDimensionNumbers(
            offset_dims=(), collapsed_slice_dims=(0,), start_index_map=(0,)
        )

        def _perm(v, idx):
            return lax.gather(
                v, idx[:, None], dnums, (1,),
                mode=lax.GatherScatterMode.PROMISE_IN_BOUNDS,
            )

        def start(j, s):
            rg = wid + j * NW

            @pl.when(rg < RG)
            def _():
                for eg in range(eg_n):
                    pltpu.async_copy(
                        tab4_hbm.at[eg, rg],
                        tile_v[s].at[pl.ds(eg * 8, 8)],
                        sem_i[s],
                    )

        def wait_out(s):
            pltpu.make_async_copy(out_hbm.at[pl.ds(0, 128)], lin_v[s], sem_o[s]).wait()

        start(0, 0)

        def body(j, carry):
            for s in (0, 1):
                jj = j * 2 + s
                rg = wid + jj * NW

                @pl.when(jj + 1 < n_mine)
                def _pf():
                    @pl.when(jj >= 1)
                    def _():
                        wait_out(1 - s)

                    start(jj + 1, 1 - s)

                @pl.when(rg < RG)
                def _work():
                    pltpu.make_async_copy(
                        tab4_hbm.at[0, 0], tile_v[s].at[pl.ds(0, 8)], sem_i[s]
                    ).wait()
                    pltpu.make_async_copy(
                        tab4_hbm.at[0, 0], tile_v[s].at[pl.ds(8, 8)], sem_i[s]
                    ).wait()

                    for c in range(8):
                        vs = [tile_v[s][r, pl.ds(c * L, L)] for r in range(D)]
                        for k in (1, 2, 4, 8):
                            nvs = list(vs)
                            km = kmask[k]
                            for a0 in range(L):
                                if a0 & k:
                                    continue
                                p = a0 | k
                                a, b = vs[a0], vs[p]
                                nvs[a0] = jnp.where(km, a, _perm(b, rot_r_idx[k]))
                                nvs[p] = jnp.where(km, _perm(a, rot_l_idx[k]), b)
                            vs = nvs
                        for r in range(L):
                            lin_v[s][c * L + r] = vs[r]

                    pltpu.async_copy(
                        lin_v[s], out_hbm.at[pl.ds(rg * 128, 128)], sem_o[s]
                    )
            return carry

        lax.fori_loop(0, (n_mine + 1) // 2, body, 0)

        @pl.when(n_mine >= 2)
        def _():
            wait_out(1)

        @pl.when(n_mine >= 1)
        def _():
            wait_out(0)

    return lin


def _make_sc_embed(TU, B, D, T, TIME):
    n_iter = (TU + NW - 1) // NW
    eg_n, bg_n = D // 8, B // 128
    ug_n = (TIME + 7) // 8
    mesh = plsc.VectorSubcoreMesh(core_axis_name="c", subcore_axis_name="s")

    @functools.partial(
        pl.kernel,
        mesh=mesh,
        compiler_params=pltpu.CompilerParams(use_tc_tiling_on_sc=False),
        out_type=[
            jax.ShapeDtypeStruct((TU, eg_n, bg_n, 8, 128), jnp.float32),
            jax.ShapeDtypeStruct((T, ug_n, bg_n, 8, 128), jnp.float32),
        ],
        scratch_types=[
            pltpu.VMEM((bg_n, 128), jnp.int32),
            pltpu.VMEM((bg_n, 128), jnp.int32),
            pltpu.VMEM((B, D), jnp.float32),
            pltpu.VMEM((B, D), jnp.float32),
            pltpu.VMEM((eg_n, bg_n, 8, 128), jnp.float32),
            pltpu.VMEM((eg_n, bg_n, 8, 128), jnp.float32),
            pltpu.VMEM((bg_n, 128), jnp.float32),
            pltpu.VMEM((bg_n, 128), jnp.float32),
            pltpu.SemaphoreType.DMA,
            pltpu.SemaphoreType.DMA,
            pltpu.SemaphoreType.DMA,
            pltpu.SemaphoreType.DMA,
        ],
    )
    def sc_embed(
        idx_hbm, table_hbm, out_hbm, mask_hbm,
        idx0, idx1, rows0, rows1, pb0, pb1, msk0, msk1,
        sg0, sg1, so0, so1,
    ):
        idx_v = [idx0, idx1]
        rows_v = [rows0, rows1]
        pb_v = [pb0, pb1]
        mask_v = [msk0, msk1]
        sem_g = [sg0, sg1]
        sem_o = [so0, so1]
        wid = lax.axis_index("s") * 2 + lax.axis_index("c")
        n_mine = (TU - wid + NW - 1) // NW

        lane = lax.iota(jnp.int32, L)
        kmask = {k: (lane & k) == 0 for k in (1, 2, 4, 8)}
        rot_r_idx = {k: (lane - k) & (L - 1) for k in (1, 2, 4, 8)}
        rot_l_idx = {k: (lane + k) & (L - 1) for k in (1, 2, 4, 8)}

        def _perm(v, idx):
            dnums = lax.GatherDimensionNumbers(
                offset_dims=(), collapsed_slice_dims=(0,), start_index_map=(0,)
            )
            return lax.gather(
                v, idx[:, None], dnums, (1,),
                mode=lax.GatherScatterMode.PROMISE_IN_BOUNDS,
            )

        def start_unit(j, s):
            tu = wid + j * NW

            @pl.when(tu < TU)
            def _():
                tt = tu // TIME
                u = tu % TIME
                pltpu.sync_copy(idx_hbm.at[tt, u // 8, :, u % 8, :], idx_v[s])
                for bg in range(bg_n):
                    pltpu.async_copy(
                        table_hbm.at[idx_v[s].at[bg]],
                        rows_v[s].at[pl.ds(bg * 128, 128)],
                        sem_g[s],
                    )

        def wait_out(s):
            pltpu.make_async_copy(out_hbm.at[0], pb_v[s], sem_o[s]).wait()
            pltpu.make_async_copy(mask_hbm.at[0, 0, :, 0, :], mask_v[s], sem_o[s]).wait()

        start_unit(0, 0)

        def unit_body(j, carry):
            for s in (0, 1):
                jj = j * 2 + s
                tu = wid + jj * NW

                @pl.when(jj + 1 < n_mine)
                def _pf():
                    @pl.when(jj >= 1)
                    def _():
                        wait_out(1 - s)

                    start_unit(jj + 1, 1 - s)

                @pl.when(tu < TU)
                def _work():
                    pltpu.make_async_copy(
                        table_hbm.at[pl.ds(0, B)], rows_v[s], sem_g[s]
                    ).wait()

                    def blk(i, c):
                        b0 = i * L
                        bgi = i >> 3
                        off = (i & 7) * L
                        iv = idx_v[s][bgi, pl.ds(off, L)]
                        m = jnp.where(iv == 0, 0.0, 1.0)
                        mask_v[s][bgi, pl.ds(off, L)] = m

                        vs = [rows_v[s][b0 + r] for r in range(L)]
                        for k in (1, 2, 4, 8):
                            nvs = list(vs)
                            km = kmask[k]
                            for a0 in range(L):
                                if a0 & k:
                                    continue
                                p = a0 | k
                                a, b = vs[a0], vs[p]
                                nvs[a0] = jnp.where(km, a, _perm(b, rot_r_idx[k]))
                                nvs[p] = jnp.where(km, _perm(a, rot_l_idx[k]), b)
                            vs = nvs

                        for e in range(D):
                            pb_v[s][e // 8, bgi, e % 8, pl.ds(off, L)] = vs[e] * m
                        return c

                    lax.fori_loop(0, B // L, blk, 0)

                    tt = tu // TIME
                    u = tu % TIME
                    pltpu.async_copy(pb_v[s], out_hbm.at[tu], sem_o[s])
                    pltpu.async_copy(
                        mask_v[s], mask_hbm.at[tt, u // 8, :, u % 8, :], sem_o[s]
                    )
            return carry

        lax.fori_loop(0, (n_iter + 1) // 2, unit_body, 0)

        @pl.when(n_mine >= 2)
        def _():
            wait_out(1)

        wait_out(0)

    return sc_embed


def kernel(x, table):
    b, t, time = x.shape
    v, d = table.shape
    tu = t * time
    up = (-time) % 8
    tp = time + up
    xq = jnp.pad(x.astype(jnp.int32), ((0, 0), (0, 0), (0, up)))
    xk = (
        jnp.transpose(xq, (1, 2, 0))
        .reshape(t, tp // 8, 8, b // 128, 128)
        .transpose(0, 1, 3, 2, 4)
    )
    vp = (-v) % 128
    v4 = v + vp
    tq = jnp.pad(table, ((0, vp), (0, 0)))
    tab4 = (
        jnp.transpose(tq, (1, 0))
        .reshape(d // 8, 8, v4 // 128, 128)
        .transpose(0, 2, 1, 3)
    )
    (tab_lin,) = _make_table_linearize(v4, d)(tab4)
    A, maskM = _make_sc_embed(tu, b, d, t, time)(xk, tab_lin)
    emb = (
        A.reshape(t, time, d // 8, b // 128, 8, 128)
        .transpose(3, 5, 0, 1, 2, 4)
        .reshape(b, t, time, d)
    )
    mask = (
        maskM.transpose(0, 1, 3, 2, 4)
        .reshape(t, tp, b)
        .transpose(2, 0, 1)[:, :, :time]
    )
    return emb, mask
